# baseline (speedup 1.0000x reference)
import jax
import jax.numpy as jnp
from jax import lax
from jax.experimental import pallas as pl
from jax.experimental.pallas import tpu as pltpu
from jax.experimental.pallas import tpu_sc as plsc

NC = 2
NS = 16
L = 16
NW = NC * NS
CH = 16
NIN = 4
NOUT = 2


def _emb_body(idx_hbm, table_hbm, out_hbm, idx_v, in0, in1, in2, in3,
              out0, out1, gs0, gs1, gs2, gs3, os0, os1):
    s_len = idx_hbm.shape[1]
    b_per_w = idx_v.shape[0]
    d = in0.shape[1]
    n_chunks = b_per_w // CH
    wid = lax.axis_index("s") * NC + lax.axis_index("c")
    flat = wid * b_per_w
    row = flat // s_len
    col = pl.multiple_of(flat % s_len, 8)
    pltpu.sync_copy(idx_hbm.at[row, pl.ds(col, b_per_w)], idx_v)

    ins = (in0, in1, in2, in3)
    outs = (out0, out1)
    gsems = (gs0, gs1, gs2, gs3)
    osems = (os0, os1)

    def g_issue(c, b):
        off = pl.multiple_of(c * CH, 8)
        pltpu.async_copy(table_hbm.at[idx_v.at[pl.ds(off, CH)]], ins[b],
                         gsems[b])

    def g_wait(b):
        pltpu.make_async_copy(table_hbm.at[idx_v.at[pl.ds(0, CH)]], ins[b],
                              gsems[b]).wait()

    def o_issue(c, b):
        off = pl.multiple_of(col + c * CH, 8)
        pltpu.async_copy(outs[b], out_hbm.at[row, pl.ds(off, CH)], osems[b])

    def o_wait(b):
        pltpu.make_async_copy(outs[b], out_hbm.at[0, pl.ds(0, CH)],
                              osems[b]).wait()

    def scale(bi, bo):
        inb, outb = ins[bi], outs[bo]
        nq = d // (16 * L)

        @plsc.parallel_loop(0, CH * nq, 1)
        def sq(q):
            r = q // nq
            qb = (q % nq) * (16 * L)
            for j in range(16):
                sl = pl.ds(qb + j * L, L)
                outb[r, sl] = inb[r, sl] * 32.0

    g_issue(0, 0)
    g_issue(1, 1)
    g_issue(2, 2)

    def grp(g, carry):
        for k in range(NIN):
            c = NIN * g + k
            bi = k % NIN
            bo = k % NOUT

            g_wait(bi)

            @pl.when(c + 3 < n_chunks)
            def _():
                g_issue(c + 3, (k + 3) % NIN)

            @pl.when(c >= NOUT)
            def _():
                o_wait(bo)

            scale(bi, bo)
            o_issue(c, bo)
        return carry

    lax.fori_loop(0, n_chunks // NIN, grp, 0)

    for b in range(NOUT):
        o_wait(b)


def kernel(tokens_ids, table):
    b, s = tokens_ids.shape
    v, d = table.shape
    n = b * s
    idx = tokens_ids.astype(jnp.int32)
    b_per_w = n // NW

    mesh = plsc.VectorSubcoreMesh(core_axis_name="c", subcore_axis_name="s")
    f = pl.kernel(
        _emb_body,
        out_type=jax.ShapeDtypeStruct((b, s, d), jnp.float32),
        mesh=mesh,
        scratch_types=[
            pltpu.VMEM((b_per_w,), jnp.int32),
            pltpu.VMEM((CH, d), jnp.float32),
            pltpu.VMEM((CH, d), jnp.float32),
            pltpu.VMEM((CH, d), jnp.float32),
            pltpu.VMEM((CH, d), jnp.float32),
            pltpu.VMEM((CH, d), jnp.float32),
            pltpu.VMEM((CH, d), jnp.float32),
            pltpu.SemaphoreType.DMA,
            pltpu.SemaphoreType.DMA,
            pltpu.SemaphoreType.DMA,
            pltpu.SemaphoreType.DMA,
            pltpu.SemaphoreType.DMA,
            pltpu.SemaphoreType.DMA,
        ],
    )
    return f(idx, table)


# speedup vs baseline: 1.0040x; 1.0040x over previous
import jax
import jax.numpy as jnp
from jax import lax
from jax.experimental import pallas as pl
from jax.experimental.pallas import tpu as pltpu
from jax.experimental.pallas import tpu_sc as plsc

NC = 2
NS = 16
L = 16
NW = NC * NS
CH = 16
NIN = 4
NOUT = 2


def _emb_body(idx_hbm, table_hbm, out_hbm, idx_v, in0, in1, in2, in3,
              out0, out1, gs0, gs1, gs2, gs3, os0, os1):
    s_len = idx_hbm.shape[1]
    b_per_w = idx_v.shape[0]
    d = in0.shape[1]
    n_chunks = b_per_w // CH
    wid = lax.axis_index("s") * NC + lax.axis_index("c")
    flat = wid * b_per_w
    row = flat // s_len
    col = pl.multiple_of(flat % s_len, 8)
    pltpu.sync_copy(idx_hbm.at[row, pl.ds(col, b_per_w)], idx_v)

    ins = (in0, in1, in2, in3)
    outs = (out0, out1)
    gsems = (gs0, gs1, gs2, gs3)
    osems = (os0, os1)

    def g_issue(c, b):
        off = pl.multiple_of(c * CH, 8)
        pltpu.async_copy(table_hbm.at[idx_v.at[pl.ds(off, CH)]], ins[b],
                         gsems[b])

    def g_wait(b):
        pltpu.make_async_copy(table_hbm.at[idx_v.at[pl.ds(0, CH)]], ins[b],
                              gsems[b]).wait()

    def o_issue(c, b):
        off = pl.multiple_of(col + c * CH, 8)
        pltpu.async_copy(outs[b], out_hbm.at[row, pl.ds(off, CH)], osems[b])

    def o_wait(b):
        pltpu.make_async_copy(outs[b], out_hbm.at[0, pl.ds(0, CH)],
                              osems[b]).wait()

    def scale(bi, bo):
        inb, outb = ins[bi], outs[bo]
        nq = d // (8 * L)

        @plsc.parallel_loop(0, CH * nq, 1)
        def sq(q):
            r = q // nq
            qb = (q % nq) * (8 * L)
            for j in range(8):
                sl = pl.ds(qb + j * L, L)
                outb[r, sl] = inb[r, sl] * 32.0

    g_issue(0, 0)
    g_issue(1, 1)
    g_issue(2, 2)

    def grp(g, carry):
        for k in range(NIN):
            c = NIN * g + k
            bi = k % NIN
            bo = k % NOUT

            g_wait(bi)

            @pl.when(c + 3 < n_chunks)
            def _():
                g_issue(c + 3, (k + 3) % NIN)

            @pl.when(c >= NOUT)
            def _():
                o_wait(bo)

            scale(bi, bo)
            o_issue(c, bo)
        return carry

    lax.fori_loop(0, n_chunks // NIN, grp, 0)

    for b in range(NOUT):
        o_wait(b)


def kernel(tokens_ids, table):
    b, s = tokens_ids.shape
    v, d = table.shape
    n = b * s
    idx = tokens_ids.astype(jnp.int32)
    b_per_w = n // NW

    mesh = plsc.VectorSubcoreMesh(core_axis_name="c", subcore_axis_name="s")
    f = pl.kernel(
        _emb_body,
        out_type=jax.ShapeDtypeStruct((b, s, d), jnp.float32),
        mesh=mesh,
        scratch_types=[
            pltpu.VMEM((b_per_w,), jnp.int32),
            pltpu.VMEM((CH, d), jnp.float32),
            pltpu.VMEM((CH, d), jnp.float32),
            pltpu.VMEM((CH, d), jnp.float32),
            pltpu.VMEM((CH, d), jnp.float32),
            pltpu.VMEM((CH, d), jnp.float32),
            pltpu.VMEM((CH, d), jnp.float32),
            pltpu.SemaphoreType.DMA,
            pltpu.SemaphoreType.DMA,
            pltpu.SemaphoreType.DMA,
            pltpu.SemaphoreType.DMA,
            pltpu.SemaphoreType.DMA,
            pltpu.SemaphoreType.DMA,
        ],
    )
    return f(idx, table)
